# Initial kernel scaffold; baseline (speedup 1.0000x reference)
import jax
import jax.numpy as jnp
from jax.experimental import pallas as pl


def kernel(x, edge_index, W_in, b_in, g0_W, g0_al, g0_ar, g0_b, g1_W, g1_al, g1_ar, g1_b, sa_Wq, sa_bq, sa_Wk, sa_bk, sa_Wv, sa_bv, sa_Wo, sa_bo, ca_Wq, ca_bq, ca_Wk, ca_bk, ca_Wv, ca_bv, ca_Wo, ca_bo, ff_W1, ff_b1, ff_W2, ff_b2, W_pred, b_pred, ln1_g, ln1_b, ln2_g, ln2_b, ln3_g, ln3_b):
    raise NotImplementedError("write your pallas kernel here")


# speedup vs baseline: 21.5078x; 21.5078x over previous
import functools

import jax
import jax.numpy as jnp
import numpy as np
from jax import lax
from jax.experimental import pallas as pl
from jax.experimental.pallas import tpu as pltpu
from jax.experimental.pallas import tpu_sc as plsc

_N = 10000
_E = 160000
_D = 128
_H = 8
_DH = 16
_FF = 2048
_OUT = 64

_B12 = 1000
_B3 = 400
_K = 128
_NSUB = 16
_NCORE = 2
_HALF = _N // _NCORE
_DEN0 = _HALF
_LOC = 5632
_ROWS_PER_SUB = _LOC // _NSUB
_ZR = 16
_NCHUNK = _E // _K

_f32 = jnp.float32

_M8_NP = np.zeros((_D, _H), np.float32)
for _h in range(_H):
    _M8_NP[_h * _DH:(_h + 1) * _DH, _h] = 1.0


def _mm(a, b):
    return jnp.dot(a, b, preferred_element_type=_f32)


def _k1_body(x_ref, win_ref, bin_ref, gw_ref, al_ref, ar_ref,
             h_ref, st_ref, dt_ref):
    h = _mm(x_ref[...], win_ref[...]) + bin_ref[...]
    h_ref[...] = h
    hp = _mm(h, gw_ref[...])
    el = _mm(hp, al_ref[...])
    er = _mm(hp, ar_ref[...])
    pad = jnp.zeros((hp.shape[0], _D - 2 * _H), _f32)
    st_ref[...] = jnp.concatenate([hp, el, el, pad], axis=1)
    dt_ref[...] = jnp.concatenate([er, er, pad], axis=1)


def _run_k1(x, W_in, b_in, gW, Al, Ar):
    grid = (_N // _B12,)
    blk = lambda i: (i, 0)
    full = lambda i: (0, 0)
    return pl.pallas_call(
        _k1_body,
        grid=grid,
        in_specs=[
            pl.BlockSpec((_B12, _D), blk),
            pl.BlockSpec((_D, _D), full),
            pl.BlockSpec((1, _D), full),
            pl.BlockSpec((_D, _D), full),
            pl.BlockSpec((_D, _H), full),
            pl.BlockSpec((_D, _H), full),
        ],
        out_specs=[
            pl.BlockSpec((_B12, _D), blk),
            pl.BlockSpec((_B12, 2 * _D), blk),
            pl.BlockSpec((_B12, _D), blk),
        ],
        out_shape=[
            jax.ShapeDtypeStruct((_N, _D), _f32),
            jax.ShapeDtypeStruct((_N, 2 * _D), _f32),
            jax.ShapeDtypeStruct((_N, _D), _f32),
        ],
    )(x, W_in, b_in, gW, Al, Ar)


def _combine(num, den, st, dt, bias, m8t):
    hp_prev = st[:, :_D]
    s = st[:, _D:_D + _H] + dt[:, :_H]
    exs = jnp.exp(jnp.where(s >= 0, s, s * 0.2))
    den8 = den[:, :_H] + exs
    numt = num + _mm(exs, m8t) * hp_prev
    return numt / _mm(den8, m8t) + bias


def _k2_body(num_ref, den_ref, stp_ref, dtp_ref, bprev_ref, m8t_ref,
             gw_ref, al_ref, ar_ref, h_ref, st_ref, dt_ref):
    h = _combine(num_ref[...], den_ref[...], stp_ref[...], dtp_ref[...],
                 bprev_ref[...], m8t_ref[...])
    h_ref[...] = h
    hp = _mm(h, gw_ref[...])
    el = _mm(hp, al_ref[...])
    er = _mm(hp, ar_ref[...])
    pad = jnp.zeros((hp.shape[0], _D - 2 * _H), _f32)
    st_ref[...] = jnp.concatenate([hp, el, el, pad], axis=1)
    dt_ref[...] = jnp.concatenate([er, er, pad], axis=1)


def _run_k2(num, den, stp, dtp, bprev, m8t, gW, Al, Ar):
    grid = (_N // _B12,)
    blk = lambda i: (i, 0)
    full = lambda i: (0, 0)
    return pl.pallas_call(
        _k2_body,
        grid=grid,
        in_specs=[
            pl.BlockSpec((_B12, _D), blk),
            pl.BlockSpec((_B12, 16), blk),
            pl.BlockSpec((_B12, 2 * _D), blk),
            pl.BlockSpec((_B12, _D), blk),
            pl.BlockSpec((1, _D), full),
            pl.BlockSpec((_H, _D), full),
            pl.BlockSpec((_D, _D), full),
            pl.BlockSpec((_D, _H), full),
            pl.BlockSpec((_D, _H), full),
        ],
        out_specs=[
            pl.BlockSpec((_B12, _D), blk),
            pl.BlockSpec((_B12, 2 * _D), blk),
            pl.BlockSpec((_B12, _D), blk),
        ],
        out_shape=[
            jax.ShapeDtypeStruct((_N, _D), _f32),
            jax.ShapeDtypeStruct((_N, 2 * _D), _f32),
            jax.ShapeDtypeStruct((_N, _D), _f32),
        ],
    )(num, den, stp, dtp, bprev, m8t, gW, Al, Ar)


def _sc_body(src_hbm, dst_hbm, st_hbm, dt_hbm, acc_out,
             idx_s, idx_d, idx_n, idx2, offs, sg, dg, prod, prod2, zbuf,
             acc_sh, sem0, sem1, sem2, sem3):
    cid = lax.axis_index("c")
    sid = lax.axis_index("s")
    zero16 = jnp.zeros((16,), _f32)

    @pl.loop(0, _ZR)
    def _(i):
        @pl.loop(0, _D // 16)
        def _(j):
            zbuf[i, pl.ds(j * 16, 16)] = zero16

    base_row = sid * _ROWS_PER_SUB

    @pl.loop(0, _ROWS_PER_SUB // _ZR)
    def _(tz):
        pltpu.sync_copy(zbuf, acc_sh.at[pl.ds(base_row + tz * _ZR, _ZR)])

    plsc.subcore_barrier()

    lo = cid * _HALF

    @pl.loop(0, (_NCHUNK + _NSUB - 1) // _NSUB)
    def _(ci):
        c = sid + ci * _NSUB

        @pl.when(c < _NCHUNK)
        def _():
            base = c * _K
            pltpu.sync_copy(src_hbm.at[pl.ds(base, _K)], idx_s)
            pltpu.sync_copy(dst_hbm.at[pl.ds(base, _K)], idx_d)
            cp1 = pltpu.async_copy(st_hbm.at[idx_s], sg, sem0)
            cp2 = pltpu.async_copy(dt_hbm.at[idx_d], dg, sem1)

            @pl.loop(0, _K // 16)
            def _(j):
                v = idx_d[pl.ds(j * 16, 16)]
                lv = v - lax.broadcast(lo, (16,))
                ok = (lv >= 0) & (lv < _HALF)
                idx_n[pl.ds(j * 16, 16)] = jnp.where(ok, lv, -1)
                idx2[pl.ds(j * 16, 16)] = jnp.where(
                    ok, _DEN0 + (lv >> 3), -1)
                offs[pl.ds(j * 16, 16)] = (lv & 7) * 16

            cp1.wait()
            cp2.wait()

            @pl.loop(0, _K // 16)
            def _(j):
                off_v = offs[pl.ds(j * 16, 16)]
                for kk in range(16):
                    k = j * 16 + kk
                    a = sg[k, pl.ds(_D, 16)]
                    b = dg[k, pl.ds(0, 16)]
                    s = a + b
                    exr = jnp.exp(jnp.where(s >= 0, s, s * 0.2))
                    for h in range(_H):
                        w = exr[h]
                        prod[k, pl.ds(h * _DH, _DH)] = (
                            sg[k, pl.ds(h * _DH, _DH)]
                            * lax.broadcast(w, (_DH,)))
                    o = off_v[kk]
                    for j2 in range(_D // 16):
                        prod2[k, pl.ds(j2 * 16, 16)] = jnp.where(
                            o == j2 * 16, exr, zero16)

            cpn = pltpu.async_copy(
                prod, acc_sh.at[plsc.Indices(idx_n, ignored_value=-1)],
                sem2, add=True)
            cpd = pltpu.async_copy(
                prod2, acc_sh.at[plsc.Indices(idx2, ignored_value=-1)],
                sem3, add=True)
            cpn.wait()
            cpd.wait()

    plsc.subcore_barrier()
    pltpu.sync_copy(acc_sh.at[pl.ds(base_row, _ROWS_PER_SUB)],
                    acc_out.at[cid, pl.ds(base_row, _ROWS_PER_SUB)])


def _run_sc(src, dst, st, dt):
    mesh = plsc.VectorSubcoreMesh(core_axis_name="c", subcore_axis_name="s")
    kern = pl.kernel(
        _sc_body,
        out_type=jax.ShapeDtypeStruct((_NCORE, _LOC, _D), _f32),
        mesh=mesh,
        scratch_types=[
            pltpu.VMEM((_K,), jnp.int32),
            pltpu.VMEM((_K,), jnp.int32),
            pltpu.VMEM((_K,), jnp.int32),
            pltpu.VMEM((_K,), jnp.int32),
            pltpu.VMEM((_K,), jnp.int32),
            pltpu.VMEM((_K, 2 * _D), _f32),
            pltpu.VMEM((_K, _D), _f32),
            pltpu.VMEM((_K, _D), _f32),
            pltpu.VMEM((_K, _D), _f32),
            pltpu.VMEM((_ZR, _D), _f32),
            pltpu.VMEM_SHARED((_LOC, _D), _f32),
            pltpu.SemaphoreType.DMA,
            pltpu.SemaphoreType.DMA,
            pltpu.SemaphoreType.DMA,
            pltpu.SemaphoreType.DMA,
        ],
    )
    return kern(src, dst, st, dt)


def _k3_body(h0_ref, h1_ref, num_ref, den_ref, stp_ref, dtp_ref, bprev_ref,
             m8_ref, m8t_ref,
             saWq, sabq, saWk, sabk, saWv, sabv, saWo, sabo,
             caWq, cabq, caWk, cabk, caWv, cabv, caWo, cabo,
             ffW1, ffb1, ffW2, ffb2, Wpred, bpred,
             ln1g, ln1b, ln2g, ln2b, ln3g, ln3b,
             z_ref):
    m8 = m8_ref[...]
    m8t = m8t_ref[...]
    h2 = _combine(num_ref[...], den_ref[...], stp_ref[...], dtp_ref[...],
                  bprev_ref[...], m8t)
    t = [h0_ref[...], h1_ref[...], h2]

    def ln(x, g, b):
        mu = jnp.mean(x, axis=-1, keepdims=True)
        xc = x - mu
        v = jnp.mean(xc * xc, axis=-1, keepdims=True)
        return xc * lax.rsqrt(v + 1e-5) * g[...] + b[...]

    def attn(q_in, kv, Wq, bq, Wk, bk, Wv, bv, Wo, bo):
        qs = [_mm(xx, Wq[...]) + bq[...] for xx in q_in]
        ks = [_mm(xx, Wk[...]) + bk[...] for xx in kv]
        vs = [_mm(xx, Wv[...]) + bv[...] for xx in kv]
        out = []
        for i in range(3):
            sc = [_mm(qs[i] * ks[j], m8) * 0.25 for j in range(3)]
            mx = jnp.maximum(jnp.maximum(sc[0], sc[1]), sc[2])
            ex = [jnp.exp(sc[j] - mx) for j in range(3)]
            dn = ex[0] + ex[1] + ex[2]
            o = _mm(ex[0] / dn, m8t) * vs[0]
            o = o + _mm(ex[1] / dn, m8t) * vs[1]
            o = o + _mm(ex[2] / dn, m8t) * vs[2]
            out.append(_mm(o, Wo[...]) + bo[...])
        return out

    sa = attn(t, t, saWq, sabq, saWk, sabk, saWv, sabv, saWo, sabo)
    u = [ln(t[i] + sa[i], ln1g, ln1b) for i in range(3)]
    ca = attn(u, t, caWq, cabq, caWk, cabk, caWv, cabv, caWo, cabo)
    w = [ln(u[i] + ca[i], ln2g, ln2b) for i in range(3)]
    y = []
    for i in range(3):
        f = _mm(jax.nn.relu(_mm(w[i], ffW1[...]) + ffb1[...]), ffW2[...])
        y.append(ln(w[i] + f + ffb2[...], ln3g, ln3b))
    z = (_mm(y[0], Wpred[0:_D, :]) + _mm(y[1], Wpred[_D:2 * _D, :])
         + _mm(y[2], Wpred[2 * _D:3 * _D, :]) + bpred[...])
    z_ref[...] = z


def _run_k3(h0, h1, num, den, st1, dt1, g1_b, m8, m8t, weights):
    grid = (_N // _B3,)
    blk = lambda i: (i, 0)
    full = lambda i: (0, 0)
    wspecs = [pl.BlockSpec(w.shape, full) for w in weights]
    return pl.pallas_call(
        _k3_body,
        grid=grid,
        in_specs=[
            pl.BlockSpec((_B3, _D), blk),
            pl.BlockSpec((_B3, _D), blk),
            pl.BlockSpec((_B3, _D), blk),
            pl.BlockSpec((_B3, 16), blk),
            pl.BlockSpec((_B3, 2 * _D), blk),
            pl.BlockSpec((_B3, _D), blk),
            pl.BlockSpec((1, _D), full),
            pl.BlockSpec((_D, _H), full),
            pl.BlockSpec((_H, _D), full),
        ] + wspecs,
        out_specs=pl.BlockSpec((_B3, _OUT), blk),
        out_shape=jax.ShapeDtypeStruct((_N, _OUT), _f32),
    )(h0, h1, num, den, st1, dt1, g1_b, m8, m8t, *weights)


def kernel(x, edge_index, W_in, b_in, g0_W, g0_al, g0_ar, g0_b, g1_W, g1_al,
           g1_ar, g1_b, sa_Wq, sa_bq, sa_Wk, sa_bk, sa_Wv, sa_bv, sa_Wo,
           sa_bo, ca_Wq, ca_bq, ca_Wk, ca_bk, ca_Wv, ca_bv, ca_Wo, ca_bo,
           ff_W1, ff_b1, ff_W2, ff_b2, W_pred, b_pred, ln1_g, ln1_b, ln2_g,
           ln2_b, ln3_g, ln3_b):
    m8 = jnp.asarray(_M8_NP)
    m8t = jnp.asarray(_M8_NP.T)
    A0l = m8 * g0_al.reshape(-1)[:, None]
    A0r = m8 * g0_ar.reshape(-1)[:, None]
    A1l = m8 * g1_al.reshape(-1)[:, None]
    A1r = m8 * g1_ar.reshape(-1)[:, None]
    src = edge_index[0]
    dst = edge_index[1]

    row = lambda v: v.reshape(1, -1)

    def split_acc(a):
        num = jnp.concatenate([a[0, :_HALF], a[1, :_HALF]], axis=0)
        denp = a[:, _DEN0:].reshape(_NCORE, (_LOC - _DEN0) * 8, 16)
        den = jnp.concatenate([denp[0, :_HALF], denp[1, :_HALF]], axis=0)
        return num, den

    h0, st0, dt0 = _run_k1(x, W_in, row(b_in), g0_W, A0l, A0r)
    num0, den0 = split_acc(_run_sc(src, dst, st0, dt0))
    h1, st1, dt1 = _run_k2(num0, den0, st0, dt0, row(g0_b), m8t,
                           g1_W, A1l, A1r)
    num1, den1 = split_acc(_run_sc(src, dst, st1, dt1))
    weights = [sa_Wq, row(sa_bq), sa_Wk, row(sa_bk), sa_Wv, row(sa_bv),
               sa_Wo, row(sa_bo), ca_Wq, row(ca_bq), ca_Wk, row(ca_bk),
               ca_Wv, row(ca_bv), ca_Wo, row(ca_bo), ff_W1, row(ff_b1),
               ff_W2, row(ff_b2), W_pred, row(b_pred), row(ln1_g),
               row(ln1_b), row(ln2_g), row(ln2_b), row(ln3_g), row(ln3_b)]
    z = _run_k3(h0, h1, num1, den1, st1, dt1, row(g1_b), m8, m8t, weights)
    return z
